# initial kernel scaffold (unmeasured)
import jax
import jax.numpy as jnp
from jax import lax
from jax.experimental import pallas as pl
from jax.experimental.pallas import tpu as pltpu

N_DEV = 4


def kernel(table, idx):
    v_per, d = table.shape
    n = idx.shape[0]

    my_pos = lax.axis_index("i")
    local = idx - my_pos * v_per
    owned = (local >= 0) & (local < v_per)
    safe = jnp.where(owned, local, 0)
    partial = jnp.where(owned[:, None], table[safe], jnp.float32(0.0))

    def body(p_ref, out_ref, comm_ref, send_sems, recv_sems):
        me = lax.axis_index("i")
        left = lax.rem(me - 1 + N_DEV, N_DEV)
        right = lax.rem(me + 1, N_DEV)

        barrier_sem = pltpu.get_barrier_semaphore()
        for nbr in [left, right]:
            pl.semaphore_signal(
                barrier_sem, inc=1,
                device_id=(nbr,), device_id_type=pl.DeviceIdType.MESH,
            )
        pl.semaphore_wait(barrier_sem, 2)

        out_ref[:, :] = p_ref[:, :]
        comm_ref[0, :, :] = p_ref[:, :]

        for h in range(N_DEV - 1):
            send_slot = h % 2
            recv_slot = (h + 1) % 2
            rdma = pltpu.make_async_remote_copy(
                src_ref=comm_ref.at[send_slot],
                dst_ref=comm_ref.at[recv_slot],
                send_sem=send_sems.at[send_slot],
                recv_sem=recv_sems.at[recv_slot],
                device_id=(right,),
                device_id_type=pl.DeviceIdType.MESH,
            )
            rdma.start()
            rdma.wait()
            out_ref[:, :] = out_ref[:, :] + comm_ref[recv_slot, :, :]

    return pl.pallas_call(
        body,
        out_shape=jax.ShapeDtypeStruct((n, d), jnp.float32),
        in_specs=[pl.BlockSpec(memory_space=pltpu.VMEM)],
        out_specs=pl.BlockSpec(memory_space=pltpu.VMEM),
        scratch_shapes=[
            pltpu.VMEM((2, n, d), jnp.float32),
            pltpu.SemaphoreType.DMA((2,)),
            pltpu.SemaphoreType.DMA((2,)),
        ],
        compiler_params=pltpu.CompilerParams(collective_id=0),
    )(partial)


# baseline (device time: 393463 ns/iter reference)
import jax
import jax.numpy as jnp
from jax import lax
from jax.experimental import pallas as pl
from jax.experimental.pallas import tpu as pltpu

N_DEV = 4
K_INFLIGHT = 16


def kernel(table, idx):
    v_per, d = table.shape
    n = idx.shape[0]

    my_pos = lax.axis_index("i")
    local = idx - my_pos * v_per
    owned = (local >= 0) & (local < v_per)
    safe = jnp.clip(local, 0, v_per - 1).astype(jnp.int32)
    mask = owned.astype(jnp.float32)[:, None]

    def body(table_ref, idx_ref, mask_ref, out_ref,
             comm_ref, gather_sems, send_sems, recv_sems):
        me = lax.axis_index("i")
        left = lax.rem(me - 1 + N_DEV, N_DEV)
        right = lax.rem(me + 1, N_DEV)

        barrier_sem = pltpu.get_barrier_semaphore()
        for nbr in [left, right]:
            pl.semaphore_signal(
                barrier_sem, inc=1,
                device_id=(nbr,), device_id_type=pl.DeviceIdType.MESH,
            )
        pl.semaphore_wait(barrier_sem, 2)

        def gather_copy(i):
            return pltpu.make_async_copy(
                table_ref.at[pl.ds(idx_ref[i], 1), :],
                out_ref.at[pl.ds(i, 1), :],
                gather_sems.at[lax.rem(i, K_INFLIGHT)],
            )

        def issue(i, carry):
            @pl.when(i >= K_INFLIGHT)
            def _():
                gather_copy(i - K_INFLIGHT).wait()
            gather_copy(i).start()
            return carry

        lax.fori_loop(0, n, issue, 0)

        def drain(i, carry):
            gather_copy(i).wait()
            return carry

        lax.fori_loop(n - K_INFLIGHT, n, drain, 0)

        masked = out_ref[:, :] * mask_ref[:, :]
        out_ref[:, :] = masked
        comm_ref[0, :, :] = masked

        for h in range(N_DEV - 1):
            send_slot = h % 2
            recv_slot = (h + 1) % 2
            rdma = pltpu.make_async_remote_copy(
                src_ref=comm_ref.at[send_slot],
                dst_ref=comm_ref.at[recv_slot],
                send_sem=send_sems.at[send_slot],
                recv_sem=recv_sems.at[recv_slot],
                device_id=(right,),
                device_id_type=pl.DeviceIdType.MESH,
            )
            rdma.start()
            rdma.wait()
            out_ref[:, :] = out_ref[:, :] + comm_ref[recv_slot, :, :]

    return pl.pallas_call(
        body,
        out_shape=jax.ShapeDtypeStruct((n, d), jnp.float32),
        in_specs=[
            pl.BlockSpec(memory_space=pltpu.MemorySpace.HBM),
            pl.BlockSpec(memory_space=pltpu.SMEM),
            pl.BlockSpec(memory_space=pltpu.VMEM),
        ],
        out_specs=pl.BlockSpec(memory_space=pltpu.VMEM),
        scratch_shapes=[
            pltpu.VMEM((2, n, d), jnp.float32),
            pltpu.SemaphoreType.DMA((K_INFLIGHT,)),
            pltpu.SemaphoreType.DMA((2,)),
            pltpu.SemaphoreType.DMA((2,)),
        ],
        compiler_params=pltpu.CompilerParams(collective_id=0),
    )(table, safe, mask)


# device time: 262812 ns/iter; 1.4971x vs baseline; 1.4971x over previous
import jax
import jax.numpy as jnp
from jax import lax
from jax.experimental import pallas as pl
from jax.experimental.pallas import tpu as pltpu

N_DEV = 4
K_INFLIGHT = 16


def kernel(table, idx):
    v_per, d = table.shape
    n = idx.shape[0]

    my_pos = lax.axis_index("i")
    local = idx - my_pos * v_per
    owned = (local >= 0) & (local < v_per)
    safe = jnp.clip(local, 0, v_per - 1).astype(jnp.int32)
    mask = owned.astype(jnp.float32)[:, None]

    def body(table_ref, idx_ref, mask_ref, out_ref,
             comm_ref, gather_sems, send_sems, recv_sems):
        me = lax.axis_index("i")
        left = lax.rem(me - 1 + N_DEV, N_DEV)
        right = lax.rem(me + 1, N_DEV)

        barrier_sem = pltpu.get_barrier_semaphore()
        for nbr in [left, right]:
            pl.semaphore_signal(
                barrier_sem, inc=1,
                device_id=(nbr,), device_id_type=pl.DeviceIdType.MESH,
            )
        pl.semaphore_wait(barrier_sem, 2)

        def gather_copy(i):
            return pltpu.make_async_copy(
                table_ref.at[pl.ds(idx_ref[i], 1), :],
                out_ref.at[pl.ds(i, 1), :],
                gather_sems.at[lax.rem(i, K_INFLIGHT)],
            )

        def issue(i, carry):
            @pl.when(i >= K_INFLIGHT)
            def _():
                gather_copy(i - K_INFLIGHT).wait()
            gather_copy(i).start()
            return carry

        lax.fori_loop(0, n, issue, 0)

        def drain(i, carry):
            gather_copy(i).wait()
            return carry

        lax.fori_loop(n - K_INFLIGHT, n, drain, 0)

        out_ref[:, :] = out_ref[:, :] * mask_ref[:, :]

        C = n // N_DEV

        for s in range(N_DEV - 1):
            slot = s % 2
            send_c = lax.rem(me - s + N_DEV, N_DEV)
            recv_c = lax.rem(me - s - 1 + N_DEV, N_DEV)
            rdma = pltpu.make_async_remote_copy(
                src_ref=out_ref.at[pl.ds(send_c * C, C), :],
                dst_ref=comm_ref.at[slot],
                send_sem=send_sems.at[slot],
                recv_sem=recv_sems.at[slot],
                device_id=(right,),
                device_id_type=pl.DeviceIdType.MESH,
            )
            rdma.start()
            rdma.wait()
            out_ref[pl.ds(recv_c * C, C), :] = (
                out_ref[pl.ds(recv_c * C, C), :] + comm_ref[slot, :, :]
            )

        for s in range(N_DEV - 1):
            slot = (N_DEV - 1 + s) % 2
            send_c = lax.rem(me + 1 - s + 2 * N_DEV, N_DEV)
            rdma = pltpu.make_async_remote_copy(
                src_ref=out_ref.at[pl.ds(send_c * C, C), :],
                dst_ref=out_ref.at[pl.ds(send_c * C, C), :],
                send_sem=send_sems.at[slot],
                recv_sem=recv_sems.at[slot],
                device_id=(right,),
                device_id_type=pl.DeviceIdType.MESH,
            )
            rdma.start()
            rdma.wait()

    return pl.pallas_call(
        body,
        out_shape=jax.ShapeDtypeStruct((n, d), jnp.float32),
        in_specs=[
            pl.BlockSpec(memory_space=pltpu.MemorySpace.HBM),
            pl.BlockSpec(memory_space=pltpu.SMEM),
            pl.BlockSpec(memory_space=pltpu.VMEM),
        ],
        out_specs=pl.BlockSpec(memory_space=pltpu.VMEM),
        scratch_shapes=[
            pltpu.VMEM((2, n // N_DEV, d), jnp.float32),
            pltpu.SemaphoreType.DMA((K_INFLIGHT,)),
            pltpu.SemaphoreType.DMA((2,)),
            pltpu.SemaphoreType.DMA((2,)),
        ],
        compiler_params=pltpu.CompilerParams(collective_id=0),
    )(table, safe, mask)


# device time: 195564 ns/iter; 2.0119x vs baseline; 1.3439x over previous
import jax
import jax.numpy as jnp
from jax import lax
from jax.experimental import pallas as pl
from jax.experimental.pallas import tpu as pltpu

N_DEV = 4
K_INFLIGHT = 16


def kernel(table, idx):
    v_per, d = table.shape
    n = idx.shape[0]

    my_pos = lax.axis_index("i")
    local = idx - my_pos * v_per
    owned = (local >= 0) & (local < v_per)
    safe = jnp.clip(local, 0, v_per - 1).astype(jnp.int32)
    mask = owned.astype(jnp.float32)[:, None]

    def body(table_ref, idx_ref, mask_ref, out_ref,
             comm_r, comm_l, gather_sems,
             send_sems_r, recv_sems_r, send_sems_l, recv_sems_l):
        me = lax.axis_index("i")
        left = lax.rem(me - 1 + N_DEV, N_DEV)
        right = lax.rem(me + 1, N_DEV)

        barrier_sem = pltpu.get_barrier_semaphore()
        for nbr in [left, right]:
            pl.semaphore_signal(
                barrier_sem, inc=1,
                device_id=(nbr,), device_id_type=pl.DeviceIdType.MESH,
            )
        pl.semaphore_wait(barrier_sem, 2)

        def gather_copy(i):
            return pltpu.make_async_copy(
                table_ref.at[pl.ds(idx_ref[i], 1), :],
                out_ref.at[pl.ds(i, 1), :],
                gather_sems.at[lax.rem(i, K_INFLIGHT)],
            )

        def issue(i, carry):
            @pl.when(i >= K_INFLIGHT)
            def _():
                gather_copy(i - K_INFLIGHT).wait()
            gather_copy(i).start()
            return carry

        lax.fori_loop(0, n, issue, 0)

        def drain(i, carry):
            gather_copy(i).wait()
            return carry

        lax.fori_loop(n - K_INFLIGHT, n, drain, 0)

        out_ref[:, :] = out_ref[:, :] * mask_ref[:, :]

        C = n // N_DEV
        H = d // 2

        for s in range(N_DEV - 1):
            slot = s % 2
            send_cr = lax.rem(me - s + N_DEV, N_DEV)
            recv_cr = lax.rem(me - s - 1 + N_DEV, N_DEV)
            send_cl = lax.rem(me + s, N_DEV)
            recv_cl = lax.rem(me + s + 1, N_DEV)
            rdma_r = pltpu.make_async_remote_copy(
                src_ref=out_ref.at[pl.ds(send_cr * C, C), pl.ds(0, H)],
                dst_ref=comm_r.at[slot],
                send_sem=send_sems_r.at[slot],
                recv_sem=recv_sems_r.at[slot],
                device_id=(right,),
                device_id_type=pl.DeviceIdType.MESH,
            )
            rdma_l = pltpu.make_async_remote_copy(
                src_ref=out_ref.at[pl.ds(send_cl * C, C), pl.ds(H, H)],
                dst_ref=comm_l.at[slot],
                send_sem=send_sems_l.at[slot],
                recv_sem=recv_sems_l.at[slot],
                device_id=(left,),
                device_id_type=pl.DeviceIdType.MESH,
            )
            rdma_r.start()
            rdma_l.start()
            rdma_r.wait()
            rdma_l.wait()
            out_ref[pl.ds(recv_cr * C, C), pl.ds(0, H)] = (
                out_ref[pl.ds(recv_cr * C, C), pl.ds(0, H)] + comm_r[slot, :, :]
            )
            out_ref[pl.ds(recv_cl * C, C), pl.ds(H, H)] = (
                out_ref[pl.ds(recv_cl * C, C), pl.ds(H, H)] + comm_l[slot, :, :]
            )

        for s in range(N_DEV - 1):
            slot = (N_DEV - 1 + s) % 2
            send_cr = lax.rem(me + 1 - s + 2 * N_DEV, N_DEV)
            send_cl = lax.rem(me - 1 + s + N_DEV, N_DEV)
            rdma_r = pltpu.make_async_remote_copy(
                src_ref=out_ref.at[pl.ds(send_cr * C, C), pl.ds(0, H)],
                dst_ref=out_ref.at[pl.ds(send_cr * C, C), pl.ds(0, H)],
                send_sem=send_sems_r.at[slot],
                recv_sem=recv_sems_r.at[slot],
                device_id=(right,),
                device_id_type=pl.DeviceIdType.MESH,
            )
            rdma_l = pltpu.make_async_remote_copy(
                src_ref=out_ref.at[pl.ds(send_cl * C, C), pl.ds(H, H)],
                dst_ref=out_ref.at[pl.ds(send_cl * C, C), pl.ds(H, H)],
                send_sem=send_sems_l.at[slot],
                recv_sem=recv_sems_l.at[slot],
                device_id=(left,),
                device_id_type=pl.DeviceIdType.MESH,
            )
            rdma_r.start()
            rdma_l.start()
            rdma_r.wait()
            rdma_l.wait()

    return pl.pallas_call(
        body,
        out_shape=jax.ShapeDtypeStruct((n, d), jnp.float32),
        in_specs=[
            pl.BlockSpec(memory_space=pltpu.MemorySpace.HBM),
            pl.BlockSpec(memory_space=pltpu.SMEM),
            pl.BlockSpec(memory_space=pltpu.VMEM),
        ],
        out_specs=pl.BlockSpec(memory_space=pltpu.VMEM),
        scratch_shapes=[
            pltpu.VMEM((2, n // N_DEV, d // 2), jnp.float32),
            pltpu.VMEM((2, n // N_DEV, d // 2), jnp.float32),
            pltpu.SemaphoreType.DMA((K_INFLIGHT,)),
            pltpu.SemaphoreType.DMA((2,)),
            pltpu.SemaphoreType.DMA((2,)),
            pltpu.SemaphoreType.DMA((2,)),
            pltpu.SemaphoreType.DMA((2,)),
        ],
        compiler_params=pltpu.CompilerParams(collective_id=0),
    )(table, safe, mask)


# device time: 112857 ns/iter; 3.4864x vs baseline; 1.7328x over previous
import jax
import jax.numpy as jnp
from jax import lax
from jax.experimental import pallas as pl
from jax.experimental.pallas import tpu as pltpu

N_DEV = 4
K_INFLIGHT = 32


def kernel(table, idx):
    v_per, d = table.shape
    n = idx.shape[0]

    my_pos = lax.axis_index("i")
    local = idx - my_pos * v_per
    owned = (local >= 0) & (local < v_per)
    order = jnp.argsort(jnp.logical_not(owned), stable=True).astype(jnp.int32)
    lidx = jnp.clip(local, 0, v_per - 1).astype(jnp.int32)[order]
    m = jnp.sum(owned.astype(jnp.int32)).reshape((1,))
    mask = owned.astype(jnp.float32)[:, None]

    def body(table_ref, lidx_ref, pos_ref, m_ref, mask_ref, out_ref,
             comm_r, comm_l, gather_sems,
             send_sems_r, recv_sems_r, send_sems_l, recv_sems_l):
        me = lax.axis_index("i")
        left = lax.rem(me - 1 + N_DEV, N_DEV)
        right = lax.rem(me + 1, N_DEV)

        barrier_sem = pltpu.get_barrier_semaphore()
        for nbr in [left, right]:
            pl.semaphore_signal(
                barrier_sem, inc=1,
                device_id=(nbr,), device_id_type=pl.DeviceIdType.MESH,
            )
        pl.semaphore_wait(barrier_sem, 2)

        def gather_copy(j):
            return pltpu.make_async_copy(
                table_ref.at[pl.ds(lidx_ref[j], 1), :],
                out_ref.at[pl.ds(pos_ref[j], 1), :],
                gather_sems.at[lax.rem(j, K_INFLIGHT)],
            )

        def wait_slot(j):
            pltpu.make_async_copy(
                table_ref.at[pl.ds(0, 1), :],
                out_ref.at[pl.ds(0, 1), :],
                gather_sems.at[lax.rem(j, K_INFLIGHT)],
            ).wait()

        m = m_ref[0]

        def warm(j, c):
            gather_copy(j).start()
            return c

        lax.fori_loop(0, jnp.minimum(m, K_INFLIGHT), warm, 0)

        def steady(j, c):
            wait_slot(j - K_INFLIGHT)
            gather_copy(j).start()
            return c

        lax.fori_loop(K_INFLIGHT, m, steady, 0)

        def drain(j, c):
            wait_slot(j)
            return c

        lax.fori_loop(jnp.maximum(m - K_INFLIGHT, 0), m, drain, 0)

        out_ref[:, :] = jnp.where(mask_ref[:, :] > 0.5, out_ref[:, :], 0.0)

        C = n // N_DEV
        H = d // 2

        for s in range(N_DEV - 1):
            slot = s % 2
            send_cr = lax.rem(me - s + N_DEV, N_DEV)
            recv_cr = lax.rem(me - s - 1 + N_DEV, N_DEV)
            send_cl = lax.rem(me + s, N_DEV)
            recv_cl = lax.rem(me + s + 1, N_DEV)
            rdma_r = pltpu.make_async_remote_copy(
                src_ref=out_ref.at[pl.ds(send_cr * C, C), pl.ds(0, H)],
                dst_ref=comm_r.at[slot],
                send_sem=send_sems_r.at[slot],
                recv_sem=recv_sems_r.at[slot],
                device_id=(right,),
                device_id_type=pl.DeviceIdType.MESH,
            )
            rdma_l = pltpu.make_async_remote_copy(
                src_ref=out_ref.at[pl.ds(send_cl * C, C), pl.ds(H, H)],
                dst_ref=comm_l.at[slot],
                send_sem=send_sems_l.at[slot],
                recv_sem=recv_sems_l.at[slot],
                device_id=(left,),
                device_id_type=pl.DeviceIdType.MESH,
            )
            rdma_r.start()
            rdma_l.start()
            rdma_r.wait()
            rdma_l.wait()
            out_ref[pl.ds(recv_cr * C, C), pl.ds(0, H)] = (
                out_ref[pl.ds(recv_cr * C, C), pl.ds(0, H)] + comm_r[slot, :, :]
            )
            out_ref[pl.ds(recv_cl * C, C), pl.ds(H, H)] = (
                out_ref[pl.ds(recv_cl * C, C), pl.ds(H, H)] + comm_l[slot, :, :]
            )

        for s in range(N_DEV - 1):
            slot = (N_DEV - 1 + s) % 2
            send_cr = lax.rem(me + 1 - s + 2 * N_DEV, N_DEV)
            send_cl = lax.rem(me - 1 + s + N_DEV, N_DEV)
            rdma_r = pltpu.make_async_remote_copy(
                src_ref=out_ref.at[pl.ds(send_cr * C, C), pl.ds(0, H)],
                dst_ref=out_ref.at[pl.ds(send_cr * C, C), pl.ds(0, H)],
                send_sem=send_sems_r.at[slot],
                recv_sem=recv_sems_r.at[slot],
                device_id=(right,),
                device_id_type=pl.DeviceIdType.MESH,
            )
            rdma_l = pltpu.make_async_remote_copy(
                src_ref=out_ref.at[pl.ds(send_cl * C, C), pl.ds(H, H)],
                dst_ref=out_ref.at[pl.ds(send_cl * C, C), pl.ds(H, H)],
                send_sem=send_sems_l.at[slot],
                recv_sem=recv_sems_l.at[slot],
                device_id=(left,),
                device_id_type=pl.DeviceIdType.MESH,
            )
            rdma_r.start()
            rdma_l.start()
            rdma_r.wait()
            rdma_l.wait()

    return pl.pallas_call(
        body,
        out_shape=jax.ShapeDtypeStruct((n, d), jnp.float32),
        in_specs=[
            pl.BlockSpec(memory_space=pltpu.MemorySpace.HBM),
            pl.BlockSpec(memory_space=pltpu.SMEM),
            pl.BlockSpec(memory_space=pltpu.SMEM),
            pl.BlockSpec(memory_space=pltpu.SMEM),
            pl.BlockSpec(memory_space=pltpu.VMEM),
        ],
        out_specs=pl.BlockSpec(memory_space=pltpu.VMEM),
        scratch_shapes=[
            pltpu.VMEM((2, n // N_DEV, d // 2), jnp.float32),
            pltpu.VMEM((2, n // N_DEV, d // 2), jnp.float32),
            pltpu.SemaphoreType.DMA((K_INFLIGHT,)),
            pltpu.SemaphoreType.DMA((2,)),
            pltpu.SemaphoreType.DMA((2,)),
            pltpu.SemaphoreType.DMA((2,)),
            pltpu.SemaphoreType.DMA((2,)),
        ],
        compiler_params=pltpu.CompilerParams(collective_id=0),
    )(table, lidx, order, m, mask)
